# baseline (speedup 1.0000x reference)
import functools

import jax
import jax.numpy as jnp
from jax.experimental import pallas as pl
from jax.experimental.pallas import tpu as pltpu


def _layer1_body(x_ref, w_ref, b_ref, adj_ref, out_ref, s_ref):
    @pl.when(pl.program_id(0) == 0)
    def _():
        s_ref[...] = jnp.dot(x_ref[...], w_ref[...],
                             preferred_element_type=jnp.float32)

    acc = jnp.dot(adj_ref[...], s_ref[...], preferred_element_type=jnp.float32)
    out_ref[...] = jnp.maximum(acc + b_ref[...], 0.0)


def _layer2_body(h_ref, w_ref, b_ref, adj_ref, out_ref, s_ref):
    @pl.when(pl.program_id(0) == 0)
    def _():
        s_ref[...] = jnp.dot(h_ref[...], w_ref[...],
                             preferred_element_type=jnp.float32)

    o = jnp.dot(adj_ref[...], s_ref[...], preferred_element_type=jnp.float32)
    o = o + b_ref[...]
    m = jnp.max(o, axis=1, keepdims=True)
    shifted = o - m
    lse = jnp.log(jnp.sum(jnp.exp(shifted), axis=1, keepdims=True))
    out_ref[...] = shifted - lse


def _block_rows(n):
    for bm in (400, 200, 100, 50, 25, 8, 4, 2, 1):
        if n % bm == 0:
            return bm
    return n


def _layer_call(body, feat, w, b, adj):
    n, k = adj.shape
    fdim = feat.shape[1]
    nout = w.shape[1]
    bm = _block_rows(n)
    grid = (n // bm,)
    return pl.pallas_call(
        body,
        grid=grid,
        in_specs=[
            pl.BlockSpec((n, fdim), lambda i: (0, 0)),
            pl.BlockSpec((fdim, nout), lambda i: (0, 0)),
            pl.BlockSpec((1, nout), lambda i: (0, 0)),
            pl.BlockSpec((bm, k), lambda i: (i, 0)),
        ],
        out_specs=pl.BlockSpec((bm, nout), lambda i: (i, 0)),
        out_shape=jax.ShapeDtypeStruct((n, nout), jnp.float32),
        scratch_shapes=[pltpu.VMEM((n, nout), jnp.float32)],
    )(feat, w, b, adj)


@functools.partial(jax.jit)
def kernel(x, adj, W1, b1, W2, b2):
    h = _layer_call(_layer1_body, x, W1, b1.reshape(1, -1), adj)
    return _layer_call(_layer2_body, h, W2, b2.reshape(1, -1), adj)


# speedup vs baseline: 1.0417x; 1.0417x over previous
import functools

import jax
import jax.numpy as jnp
from jax.experimental import pallas as pl
from jax.experimental.pallas import tpu as pltpu

_BM = 400


def _body(x_ref, w1_ref, b1_ref, w2_ref, b2_ref, adj_ref, out_ref,
          s_ref, h_ref, *, nsteps):
    i = pl.program_id(0)

    @pl.when(i == 0)
    def _():
        s_ref[...] = jnp.dot(x_ref[...], w1_ref[...],
                             preferred_element_type=jnp.float32)

    @pl.when(i == nsteps)
    def _():
        s_ref[...] = jnp.dot(h_ref[...], w2_ref[...],
                             preferred_element_type=jnp.float32)

    o = jnp.dot(adj_ref[...], s_ref[...], preferred_element_type=jnp.float32)

    @pl.when(i < nsteps)
    def _():
        h_ref[pl.ds(i * _BM, _BM), :] = jnp.maximum(o + b1_ref[...], 0.0)

    @pl.when(i >= nsteps)
    def _():
        oo = o + b2_ref[...]
        shifted = oo - jnp.max(oo, axis=1, keepdims=True)
        lse = jnp.log(jnp.sum(jnp.exp(shifted), axis=1, keepdims=True))
        out_ref[...] = shifted - lse


@jax.jit
def kernel(x, adj, W1, b1, W2, b2):
    n, _ = adj.shape
    nfeat = x.shape[1]
    nhid = W1.shape[1]
    nclass = W2.shape[1]
    t = n // _BM

    return pl.pallas_call(
        functools.partial(_body, nsteps=t),
        grid=(2 * t,),
        in_specs=[
            pl.BlockSpec((n, nfeat), lambda i: (0, 0)),
            pl.BlockSpec((nfeat, nhid), lambda i: (0, 0)),
            pl.BlockSpec((1, nhid), lambda i: (0, 0)),
            pl.BlockSpec((nhid, nclass), lambda i: (0, 0)),
            pl.BlockSpec((1, nclass), lambda i: (0, 0)),
            pl.BlockSpec((_BM, n), lambda i: (i % t, 0)),
        ],
        out_specs=pl.BlockSpec(
            (_BM, nclass), lambda i: (jnp.where(i < t, 0, i - t), 0)),
        out_shape=jax.ShapeDtypeStruct((n, nclass), jnp.float32),
        scratch_shapes=[
            pltpu.VMEM((n, nhid), jnp.float32),
            pltpu.VMEM((n, nhid), jnp.float32),
        ],
        compiler_params=pltpu.CompilerParams(
            dimension_semantics=("arbitrary",),
        ),
    )(x, W1, b1.reshape(1, -1), W2, b2.reshape(1, -1), adj)
